# baseline (device time: 207826 ns/iter reference)
import jax
import jax.numpy as jnp
from jax import lax
from jax.experimental import pallas as pl
from jax.experimental.pallas import tpu as pltpu

N = 16
B = 64
D = 1024
H = 2048
G = 2
NG = N // G


def kernel(x, Win0, Wout0, Win1, Wout1, Win2, Wout2):
    xb = x.astype(jnp.bfloat16)

    def body(x_ref, w_in0, w_out0, w_in1, w_out1, w_in2, w_out2, out_ref,
             X, Pb, rbuf, accb, win_st, wout_st, winb, woutb,
             ag_s, ag_r, rs_s, rs_r, wdma):
        me = lax.axis_index("i")
        w_hbm = ((w_in0, w_out0), (w_in1, w_out1), (w_in2, w_out2))

        def wstage_start(layer):
            pltpu.make_async_copy(w_hbm[layer][0], win_st, wdma.at[0]).start()
            pltpu.make_async_copy(w_hbm[layer][1], wout_st, wdma.at[1]).start()

        def wstage_finish(layer):
            slot = layer % 2
            pltpu.make_async_copy(w_hbm[layer][0], win_st, wdma.at[0]).wait()
            pltpu.make_async_copy(w_hbm[layer][1], wout_st, wdma.at[1]).wait()
            winb[slot] = win_st[...].astype(jnp.bfloat16)
            woutb[slot] = wout_st[...].astype(jnp.bfloat16)

        wstage_start(0)

        bar = pltpu.get_barrier_semaphore()

        def bar_sig(d, c):
            j = lax.rem(me + d, N)
            pl.semaphore_signal(bar, inc=1, device_id=(j,),
                                device_id_type=pl.DeviceIdType.MESH)
            return c

        lax.fori_loop(1, N, bar_sig, 0)
        pl.semaphore_wait(bar, N - 1)

        def ag_send_all():
            for j in range(N):
                @pl.when(j != me)
                def _():
                    pltpu.make_async_remote_copy(
                        src_ref=X.at[pl.ds(me * B, B), :],
                        dst_ref=X.at[pl.ds(me * B, B), :],
                        send_sem=ag_s.at[j],
                        recv_sem=ag_r.at[me],
                        device_id=(j,),
                        device_id_type=pl.DeviceIdType.MESH,
                    ).start()

        def ag_wait_chunk(j):
            @pl.when(j != me)
            def _():
                pltpu.make_async_remote_copy(
                    src_ref=X.at[pl.ds(me * B, B), :],
                    dst_ref=X.at[pl.ds(j * B, B), :],
                    send_sem=ag_s.at[j],
                    recv_sem=ag_r.at[j],
                    device_id=(j,),
                    device_id_type=pl.DeviceIdType.MESH,
                ).wait_recv()

        def wait_send_all(sems):
            for j in range(N):
                @pl.when(j != me)
                def _():
                    pltpu.make_async_remote_copy(
                        src_ref=X.at[pl.ds(me * B, B), :],
                        dst_ref=X.at[pl.ds(me * B, B), :],
                        send_sem=sems.at[j],
                        recv_sem=ag_r.at[me],
                        device_id=(j,),
                        device_id_type=pl.DeviceIdType.MESH,
                    ).wait_send()

        X[pl.ds(me * B, B), :] = x_ref[...]
        ag_send_all()

        wstage_finish(0)
        wstage_start(1)

        def run_layer(layer):
            slot = layer % 2
            for g in range(NG):
                for j in range(g * G, (g + 1) * G):
                    ag_wait_chunk(j)
                if g == 1:
                    wait_send_all(ag_s)
                if g == 2 and layer < 2:
                    wstage_finish(layer + 1)
                    if layer < 1:
                        wstage_start(layer + 2)
                r0 = g * G * B
                hg = jnp.dot(X[r0:r0 + G * B, :], winb[slot],
                             preferred_element_type=jnp.float32)
                hg = jnp.maximum(hg, 0.0).astype(jnp.bfloat16)
                pg = jnp.dot(hg, woutb[slot],
                             preferred_element_type=jnp.float32)
                Pb[r0:r0 + G * B, :] = pg.astype(jnp.bfloat16)

                for j in range(g * G, (g + 1) * G):
                    @pl.when(j != me)
                    def _():
                        pltpu.make_async_remote_copy(
                            src_ref=Pb.at[pl.ds(j * B, B), :],
                            dst_ref=rbuf.at[me],
                            send_sem=rs_s.at[j],
                            recv_sem=rs_r.at[me],
                            device_id=(j,),
                            device_id_type=pl.DeviceIdType.MESH,
                        ).start()

            accb[...] = Pb[pl.ds(me * B, B), :].astype(jnp.float32)
            for j in range(N):
                @pl.when(j != me)
                def _():
                    pltpu.make_async_remote_copy(
                        src_ref=Pb.at[pl.ds(me * B, B), :],
                        dst_ref=rbuf.at[j],
                        send_sem=rs_s.at[j],
                        recv_sem=rs_r.at[j],
                        device_id=(j,),
                        device_id_type=pl.DeviceIdType.MESH,
                    ).wait_recv()
                    accb[...] += rbuf[j].astype(jnp.float32)

            wait_send_all(rs_s)
            X[pl.ds(me * B, B), :] = accb[...].astype(jnp.bfloat16)
            ag_send_all()

        run_layer(0)
        run_layer(1)
        run_layer(2)

        for g in range(NG):
            for j in range(g * G, (g + 1) * G):
                ag_wait_chunk(j)
            r0 = g * G * B
            out_ref[r0:r0 + G * B, :] = (
                X[r0:r0 + G * B, :].astype(jnp.float32)
            )
        wait_send_all(ag_s)

    return pl.pallas_call(
        body,
        out_shape=jax.ShapeDtypeStruct((N * B, D), jnp.float32),
        in_specs=[pl.BlockSpec(memory_space=pltpu.VMEM)]
        + [pl.BlockSpec(memory_space=pl.ANY)] * 6,
        out_specs=pl.BlockSpec(memory_space=pltpu.VMEM),
        scratch_shapes=[
            pltpu.VMEM((N * B, D), jnp.bfloat16),
            pltpu.VMEM((N * B, D), jnp.bfloat16),
            pltpu.VMEM((N, B, D), jnp.bfloat16),
            pltpu.VMEM((B, D), jnp.float32),
            pltpu.VMEM((D, H), jnp.float32),
            pltpu.VMEM((H, D), jnp.float32),
            pltpu.VMEM((2, D, H), jnp.bfloat16),
            pltpu.VMEM((2, H, D), jnp.bfloat16),
            pltpu.SemaphoreType.DMA((N,)),
            pltpu.SemaphoreType.DMA((N,)),
            pltpu.SemaphoreType.DMA((N,)),
            pltpu.SemaphoreType.DMA((N,)),
            pltpu.SemaphoreType.DMA((2,)),
        ],
        compiler_params=pltpu.CompilerParams(
            collective_id=0, vmem_limit_bytes=60 * 1024 * 1024),
    )(xb, Win0, Wout0, Win1, Wout1, Win2, Wout2)


# device time: 197539 ns/iter; 1.0521x vs baseline; 1.0521x over previous
import jax
import jax.numpy as jnp
from jax import lax
from jax.experimental import pallas as pl
from jax.experimental.pallas import tpu as pltpu

N = 16
B = 64
D = 1024
H = 2048
G = 4
NG = N // G


def kernel(x, Win0, Wout0, Win1, Wout1, Win2, Wout2):
    xb = x.astype(jnp.bfloat16)

    def body(x_ref, w_in0, w_out0, w_in1, w_out1, w_in2, w_out2, out_ref,
             X, Pb, rbuf, accb, win_st, wout_st, winb, woutb,
             ag_s, ag_r, rs_s, rs_r, wdma):
        me = lax.axis_index("i")
        w_hbm = ((w_in0, w_out0), (w_in1, w_out1), (w_in2, w_out2))

        def wstage_start(layer):
            pltpu.make_async_copy(w_hbm[layer][0], win_st, wdma.at[0]).start()
            pltpu.make_async_copy(w_hbm[layer][1], wout_st, wdma.at[1]).start()

        def wstage_finish(layer):
            slot = layer % 2
            pltpu.make_async_copy(w_hbm[layer][0], win_st, wdma.at[0]).wait()
            pltpu.make_async_copy(w_hbm[layer][1], wout_st, wdma.at[1]).wait()
            winb[slot] = win_st[...].astype(jnp.bfloat16)
            woutb[slot] = wout_st[...].astype(jnp.bfloat16)

        wstage_start(0)

        bar = pltpu.get_barrier_semaphore()

        def bar_sig(d, c):
            j = lax.rem(me + d, N)
            pl.semaphore_signal(bar, inc=1, device_id=(j,),
                                device_id_type=pl.DeviceIdType.MESH)
            return c

        lax.fori_loop(1, N, bar_sig, 0)
        pl.semaphore_wait(bar, N - 1)

        def ag_send_all():
            for j in range(N):
                @pl.when(j != me)
                def _():
                    pltpu.make_async_remote_copy(
                        src_ref=X.at[pl.ds(me * B, B), :],
                        dst_ref=X.at[pl.ds(me * B, B), :],
                        send_sem=ag_s.at[j],
                        recv_sem=ag_r.at[me],
                        device_id=(j,),
                        device_id_type=pl.DeviceIdType.MESH,
                    ).start()

        def ag_wait_chunk(j):
            @pl.when(j != me)
            def _():
                pltpu.make_async_remote_copy(
                    src_ref=X.at[pl.ds(me * B, B), :],
                    dst_ref=X.at[pl.ds(j * B, B), :],
                    send_sem=ag_s.at[j],
                    recv_sem=ag_r.at[j],
                    device_id=(j,),
                    device_id_type=pl.DeviceIdType.MESH,
                ).wait_recv()

        def wait_send_all(sems):
            for j in range(N):
                @pl.when(j != me)
                def _():
                    pltpu.make_async_remote_copy(
                        src_ref=X.at[pl.ds(me * B, B), :],
                        dst_ref=X.at[pl.ds(me * B, B), :],
                        send_sem=sems.at[j],
                        recv_sem=ag_r.at[me],
                        device_id=(j,),
                        device_id_type=pl.DeviceIdType.MESH,
                    ).wait_send()

        X[pl.ds(me * B, B), :] = x_ref[...]
        ag_send_all()

        wstage_finish(0)
        wstage_start(1)

        def run_layer(layer):
            slot = layer % 2
            for g in range(NG):
                for j in range(g * G, (g + 1) * G):
                    ag_wait_chunk(j)
                if g == 1:
                    wait_send_all(ag_s)
                if g == 2 and layer < 2:
                    wstage_finish(layer + 1)
                    if layer < 1:
                        wstage_start(layer + 2)
                r0 = g * G * B
                hg = jnp.dot(X[r0:r0 + G * B, :], winb[slot],
                             preferred_element_type=jnp.float32)
                hg = jnp.maximum(hg, 0.0).astype(jnp.bfloat16)
                pg = jnp.dot(hg, woutb[slot],
                             preferred_element_type=jnp.float32)
                Pb[r0:r0 + G * B, :] = pg.astype(jnp.bfloat16)

                for j in range(g * G, (g + 1) * G):
                    @pl.when(j != me)
                    def _():
                        pltpu.make_async_remote_copy(
                            src_ref=Pb.at[pl.ds(j * B, B), :],
                            dst_ref=rbuf.at[me],
                            send_sem=rs_s.at[j],
                            recv_sem=rs_r.at[me],
                            device_id=(j,),
                            device_id_type=pl.DeviceIdType.MESH,
                        ).start()

            accb[...] = Pb[pl.ds(me * B, B), :].astype(jnp.float32)
            for j in range(N):
                @pl.when(j != me)
                def _():
                    pltpu.make_async_remote_copy(
                        src_ref=Pb.at[pl.ds(me * B, B), :],
                        dst_ref=rbuf.at[j],
                        send_sem=rs_s.at[j],
                        recv_sem=rs_r.at[j],
                        device_id=(j,),
                        device_id_type=pl.DeviceIdType.MESH,
                    ).wait_recv()
                    accb[...] += rbuf[j].astype(jnp.float32)

            wait_send_all(rs_s)
            X[pl.ds(me * B, B), :] = accb[...].astype(jnp.bfloat16)
            ag_send_all()

        run_layer(0)
        run_layer(1)
        run_layer(2)

        for g in range(NG):
            for j in range(g * G, (g + 1) * G):
                ag_wait_chunk(j)
            r0 = g * G * B
            out_ref[r0:r0 + G * B, :] = (
                X[r0:r0 + G * B, :].astype(jnp.float32)
            )
        wait_send_all(ag_s)

    return pl.pallas_call(
        body,
        out_shape=jax.ShapeDtypeStruct((N * B, D), jnp.float32),
        in_specs=[pl.BlockSpec(memory_space=pltpu.VMEM)]
        + [pl.BlockSpec(memory_space=pl.ANY)] * 6,
        out_specs=pl.BlockSpec(memory_space=pltpu.VMEM),
        scratch_shapes=[
            pltpu.VMEM((N * B, D), jnp.bfloat16),
            pltpu.VMEM((N * B, D), jnp.bfloat16),
            pltpu.VMEM((N, B, D), jnp.bfloat16),
            pltpu.VMEM((B, D), jnp.float32),
            pltpu.VMEM((D, H), jnp.float32),
            pltpu.VMEM((H, D), jnp.float32),
            pltpu.VMEM((2, D, H), jnp.bfloat16),
            pltpu.VMEM((2, H, D), jnp.bfloat16),
            pltpu.SemaphoreType.DMA((N,)),
            pltpu.SemaphoreType.DMA((N,)),
            pltpu.SemaphoreType.DMA((N,)),
            pltpu.SemaphoreType.DMA((N,)),
            pltpu.SemaphoreType.DMA((2,)),
        ],
        compiler_params=pltpu.CompilerParams(
            collective_id=0, vmem_limit_bytes=60 * 1024 * 1024),
    )(xb, Win0, Wout0, Win1, Wout1, Win2, Wout2)
